# baseline (speedup 1.0000x reference)
import functools

import jax
import jax.numpy as jnp
from jax import lax
from jax.experimental import pallas as pl
from jax.experimental.pallas import tpu as pltpu
from jax.experimental.pallas import tpu_sc as plsc

NC = 2
NS = 16
NW = NC * NS
L = 16
D = 64
DP = 2 * D

VB = 1 << 20
HB = VB // 2
SC_CHUNK = 128

CHP = 128
NCH = 122
PER_TILE = CHP * NCH
TAIL = ((499712, 128), (499840, 128), (499968, 32))


def _sc_histogram(idx, n_idx):
    per_tile = n_idx // NW
    nstream = per_tile // SC_CHUNK
    slice_per_tile = VB // NS
    mesh = plsc.VectorSubcoreMesh(core_axis_name="c", subcore_axis_name="s")

    @functools.partial(
        pl.kernel,
        out_type=jax.ShapeDtypeStruct((NC * VB,), jnp.float32),
        mesh=mesh,
        scratch_types=[
            pltpu.VMEM((per_tile,), jnp.int32),
            pltpu.VMEM((SC_CHUNK,), jnp.float32),
            pltpu.VMEM((slice_per_tile // 4,), jnp.float32),
            pltpu.VMEM_SHARED((VB,), jnp.float32),
            pltpu.SemaphoreType.DMA,
        ],
    )
    def k(idx_hbm, out_hbm, idx_v, ones_v, zero_v, hist_sp, sem):
        core = lax.axis_index("c")
        sub = lax.axis_index("s")
        base = (core * NS + sub) * per_tile
        pltpu.sync_copy(idx_hbm.at[pl.ds(base, per_tile)], idx_v)

        def xform(kk, _):
            v = idx_v[pl.ds(kk * L, L)]
            idx_v[pl.ds(kk * L, L)] = (
                lax.shift_right_logical(v, 1) + (v & 1) * HB
            )
            return 0

        lax.fori_loop(0, per_tile // L, xform, 0, unroll=4)

        def fill_ones(kk, _):
            ones_v[pl.ds(kk * L, L)] = jnp.ones((L,), jnp.float32)
            return 0

        lax.fori_loop(0, SC_CHUNK // L, fill_ones, 0)

        def fill_zero(kk, _):
            zero_v[pl.ds(kk * L, L)] = jnp.zeros((L,), jnp.float32)
            return 0

        qtr = slice_per_tile // 4
        lax.fori_loop(0, qtr // L, fill_zero, 0, unroll=8)

        for q in range(4):
            pltpu.sync_copy(
                zero_v, hist_sp.at[pl.ds(sub * slice_per_tile + q * qtr, qtr)]
            )
        plsc.subcore_barrier()

        def fire(cc, _):
            pltpu.async_copy(
                ones_v,
                hist_sp.at[idx_v.at[pl.ds(cc * SC_CHUNK, SC_CHUNK)]],
                sem,
                add=True,
            )
            return 0

        lax.fori_loop(0, nstream, fire, 0)

        def drain(cc, _):
            pltpu.make_async_copy(
                ones_v,
                hist_sp.at[idx_v.at[pl.ds(0, SC_CHUNK)]],
                sem,
            ).wait()
            return 0

        lax.fori_loop(0, nstream, drain, 0)

        plsc.subcore_barrier()
        pltpu.sync_copy(
            hist_sp.at[pl.ds(sub * slice_per_tile, slice_per_tile)],
            out_hbm.at[pl.ds(core * VB + sub * slice_per_tile, slice_per_tile)],
        )

    return k(idx)


def _sc_matvec(counts, table_flat):
    mesh = plsc.VectorSubcoreMesh(core_axis_name="c", subcore_axis_name="s")

    @functools.partial(
        pl.kernel,
        out_type=jax.ShapeDtypeStruct((NW, D), jnp.float32),
        mesh=mesh,
        scratch_types=[
            pltpu.VMEM((2, CHP * DP), jnp.float32),
            pltpu.VMEM((2, 4, CHP), jnp.float32),
            pltpu.VMEM((D,), jnp.float32),
            pltpu.SemaphoreType.DMA((2,)),
            pltpu.SemaphoreType.DMA((2,)),
        ],
    )
    def k(cnt_hbm, tab_hbm, out_hbm, tbuf, cbuf, acc_v, tsems, csems):
        core = lax.axis_index("c")
        sub = lax.axis_index("s")
        wid = core * NS + sub
        base = wid * PER_TILE

        def start_at(p0, n, slot):
            pltpu.make_async_copy(
                tab_hbm.at[pl.ds(p0 * DP, n * DP)],
                tbuf.at[slot, pl.ds(0, n * DP)],
                tsems.at[slot],
            ).start()
            for q, off in enumerate((0, HB, VB, VB + HB)):
                pltpu.make_async_copy(
                    cnt_hbm.at[pl.ds(off + p0, n)],
                    cbuf.at[slot, q, pl.ds(0, n)],
                    csems.at[slot],
                ).start()

        def wait_at(n, slot):
            pltpu.make_async_copy(
                tab_hbm.at[pl.ds(0, n * DP)],
                tbuf.at[slot, pl.ds(0, n * DP)],
                tsems.at[slot],
            ).wait()
            for q in range(4):
                pltpu.make_async_copy(
                    cnt_hbm.at[pl.ds(0, n)],
                    cbuf.at[slot, q, pl.ds(0, n)],
                    csems.at[slot],
                ).wait()

        def group_body(g, a, slot, scale=None):
            a0, a1, a2, a3 = a
            ce = cbuf[slot, 0, pl.ds(g * L, L)] + cbuf[slot, 2, pl.ds(g * L, L)]
            co = cbuf[slot, 1, pl.ds(g * L, L)] + cbuf[slot, 3, pl.ds(g * L, L)]
            if scale is not None:
                ce = ce * scale
                co = co * scale
            for kk in range(L):
                i = (g * L + kk) * DP
                cve = jnp.full((L,), ce[kk], jnp.float32)
                cvo = jnp.full((L,), co[kk], jnp.float32)
                a0 = a0 + tbuf[slot, pl.ds(i, L)] * cve
                a1 = a1 + tbuf[slot, pl.ds(i + L, L)] * cve
                a2 = a2 + tbuf[slot, pl.ds(i + 2 * L, L)] * cve
                a3 = a3 + tbuf[slot, pl.ds(i + 3 * L, L)] * cve
                a0 = a0 + tbuf[slot, pl.ds(i + D, L)] * cvo
                a1 = a1 + tbuf[slot, pl.ds(i + D + L, L)] * cvo
                a2 = a2 + tbuf[slot, pl.ds(i + D + 2 * L, L)] * cvo
                a3 = a3 + tbuf[slot, pl.ds(i + D + 3 * L, L)] * cvo
            return (a0, a1, a2, a3)

        for b in range(2):
            start_at(base + b * CHP, CHP, b)

        def outer_body(co_i, carry):
            for b in range(2):
                c = co_i * 2 + b
                wait_at(CHP, b)
                carry = lax.fori_loop(
                    0, CHP // L, functools.partial(group_body, slot=b), carry
                )

                @pl.when(c + 2 < NCH)
                def _():
                    start_at(base + (c + 2) * CHP, CHP, b)

            return carry

        z = jnp.zeros((L,), jnp.float32)
        carry = lax.fori_loop(0, NCH // 2, outer_body, (z, z, z, z))

        m = jnp.where(wid == 0, 1.0, 0.0).astype(jnp.float32)
        mv = jnp.full((L,), m, jnp.float32)
        for p0, n in TAIL:
            start_at(p0, n, 0)
            wait_at(n, 0)
            carry = lax.fori_loop(
                0,
                n // L,
                functools.partial(group_body, slot=0, scale=mv),
                carry,
            )

        a0, a1, a2, a3 = carry
        acc_v[pl.ds(0, L)] = a0
        acc_v[pl.ds(L, L)] = a1
        acc_v[pl.ds(2 * L, L)] = a2
        acc_v[pl.ds(3 * L, L)] = a3
        pltpu.sync_copy(acc_v, out_hbm.at[wid])

    return k(counts, table_flat)


def _tc_finish(partials, w, b, n_rows, n_idx):
    def body(p_ref, w_ref, b_ref, o_ref):
        s = jnp.sum(p_ref[...], axis=0, keepdims=True) * (1.0 / n_idx)
        y = lax.dot_general(
            s, w_ref[...], (((1,), (1,)), ((), ())),
            preferred_element_type=jnp.float32,
        ) + b_ref[...][None, :]
        y = jnp.where(y >= 0, y, 0.01 * y)
        o_ref[...] = jnp.broadcast_to(y, o_ref.shape)

    return pl.pallas_call(
        body,
        out_shape=jax.ShapeDtypeStruct((n_rows, w.shape[0]), jnp.float32),
    )(partials, w, b)


@jax.jit
def kernel(x, mp_neighbors, item_table, neigh_w, neigh_b, mp):
    flat_idx = mp_neighbors.reshape(-1)
    table_flat = item_table.reshape(-1)
    counts = _sc_histogram(flat_idx, flat_idx.shape[0])
    partials = _sc_matvec(counts, table_flat)
    return _tc_finish(
        partials, neigh_w, neigh_b, x.shape[0], flat_idx.shape[0]
    )


# speedup vs baseline: 1.0709x; 1.0709x over previous
import functools

import jax
import jax.numpy as jnp
from jax import lax
from jax.experimental import pallas as pl
from jax.experimental.pallas import tpu as pltpu
from jax.experimental.pallas import tpu_sc as plsc

NC = 2
NS = 16
NW = NC * NS
L = 16
D = 64
DP = 2 * D

VB = 1 << 20
HB = VB // 2
SC_CHUNK = 128

CHP = 128
NCH = 122
PER_TILE = CHP * NCH
TAIL = ((499712, 128), (499840, 128), (499968, 32))


def _sc_histogram(idx, n_idx):
    per_tile = n_idx // NW
    nstream = per_tile // SC_CHUNK
    slice_per_tile = VB // NS
    mesh = plsc.VectorSubcoreMesh(core_axis_name="c", subcore_axis_name="s")

    @functools.partial(
        pl.kernel,
        out_type=jax.ShapeDtypeStruct((NC * VB,), jnp.float32),
        mesh=mesh,
        scratch_types=[
            pltpu.VMEM((per_tile,), jnp.int32),
            pltpu.VMEM((SC_CHUNK,), jnp.float32),
            pltpu.VMEM((slice_per_tile // 4,), jnp.float32),
            pltpu.VMEM_SHARED((VB,), jnp.float32),
            pltpu.SemaphoreType.DMA,
        ],
    )
    def k(idx_hbm, out_hbm, idx_v, ones_v, zero_v, hist_sp, sem):
        core = lax.axis_index("c")
        sub = lax.axis_index("s")
        base = (core * NS + sub) * per_tile
        pltpu.sync_copy(idx_hbm.at[pl.ds(base, per_tile)], idx_v)

        def xform(kk, _):
            v = idx_v[pl.ds(kk * L, L)]
            idx_v[pl.ds(kk * L, L)] = (
                lax.shift_right_logical(v, 1) + (v & 1) * HB
            )
            return 0

        lax.fori_loop(0, per_tile // L, xform, 0, unroll=4)

        def fill_ones(kk, _):
            ones_v[pl.ds(kk * L, L)] = jnp.ones((L,), jnp.float32)
            return 0

        lax.fori_loop(0, SC_CHUNK // L, fill_ones, 0)

        def fill_zero(kk, _):
            zero_v[pl.ds(kk * L, L)] = jnp.zeros((L,), jnp.float32)
            return 0

        qtr = slice_per_tile // 4
        lax.fori_loop(0, qtr // L, fill_zero, 0, unroll=8)

        for q in range(4):
            pltpu.sync_copy(
                zero_v, hist_sp.at[pl.ds(sub * slice_per_tile + q * qtr, qtr)]
            )
        plsc.subcore_barrier()

        def fire(cc, _):
            pltpu.async_copy(
                ones_v,
                hist_sp.at[idx_v.at[pl.ds(cc * SC_CHUNK, SC_CHUNK)]],
                sem,
                add=True,
            )
            return 0

        lax.fori_loop(0, nstream, fire, 0)

        def drain(cc, _):
            pltpu.make_async_copy(
                ones_v,
                hist_sp.at[idx_v.at[pl.ds(0, SC_CHUNK)]],
                sem,
            ).wait()
            return 0

        lax.fori_loop(0, nstream, drain, 0)

        plsc.subcore_barrier()
        pltpu.sync_copy(
            hist_sp.at[pl.ds(sub * slice_per_tile, slice_per_tile)],
            out_hbm.at[pl.ds(core * VB + sub * slice_per_tile, slice_per_tile)],
        )

    return k(idx)


def _sc_matvec(counts, table):
    mesh = plsc.VectorSubcoreMesh(core_axis_name="c", subcore_axis_name="s")

    @functools.partial(
        pl.kernel,
        out_type=jax.ShapeDtypeStruct((NW, D), jnp.float32),
        mesh=mesh,
        scratch_types=[
            pltpu.VMEM((2, 2 * CHP, D), jnp.float32),
            pltpu.VMEM((2, 4, CHP), jnp.float32),
            pltpu.VMEM((D,), jnp.float32),
            pltpu.SemaphoreType.DMA((2,)),
            pltpu.SemaphoreType.DMA((2,)),
        ],
        compiler_params=pltpu.CompilerParams(use_tc_tiling_on_sc=False),
    )
    def k(cnt_hbm, tab_hbm, out_hbm, tbuf, cbuf, acc_v, tsems, csems):
        core = lax.axis_index("c")
        sub = lax.axis_index("s")
        wid = core * NS + sub
        base = wid * PER_TILE

        def start_at(p0, n, slot):
            pltpu.make_async_copy(
                tab_hbm.at[pl.ds(p0 * 2, n * 2), :],
                tbuf.at[slot, pl.ds(0, n * 2), :],
                tsems.at[slot],
            ).start()
            for q, off in enumerate((0, HB, VB, VB + HB)):
                pltpu.make_async_copy(
                    cnt_hbm.at[pl.ds(off + p0, n)],
                    cbuf.at[slot, q, pl.ds(0, n)],
                    csems.at[slot],
                ).start()

        def wait_at(n, slot):
            pltpu.make_async_copy(
                tab_hbm.at[pl.ds(0, n * 2), :],
                tbuf.at[slot, pl.ds(0, n * 2), :],
                tsems.at[slot],
            ).wait()
            for q in range(4):
                pltpu.make_async_copy(
                    cnt_hbm.at[pl.ds(0, n)],
                    cbuf.at[slot, q, pl.ds(0, n)],
                    csems.at[slot],
                ).wait()

        def group_body(g, a, slot, scale=None):
            a0, a1, a2, a3 = a
            ce = cbuf[slot, 0, pl.ds(g * L, L)] + cbuf[slot, 2, pl.ds(g * L, L)]
            co = cbuf[slot, 1, pl.ds(g * L, L)] + cbuf[slot, 3, pl.ds(g * L, L)]
            if scale is not None:
                ce = ce * scale
                co = co * scale
            for kk in range(L):
                i = (g * L + kk) * 2
                cve = jnp.full((L,), ce[kk], jnp.float32)
                cvo = jnp.full((L,), co[kk], jnp.float32)
                a0 = a0 + tbuf[slot, i, pl.ds(0, L)] * cve
                a1 = a1 + tbuf[slot, i, pl.ds(L, L)] * cve
                a2 = a2 + tbuf[slot, i, pl.ds(2 * L, L)] * cve
                a3 = a3 + tbuf[slot, i, pl.ds(3 * L, L)] * cve
                a0 = a0 + tbuf[slot, i + 1, pl.ds(0, L)] * cvo
                a1 = a1 + tbuf[slot, i + 1, pl.ds(L, L)] * cvo
                a2 = a2 + tbuf[slot, i + 1, pl.ds(2 * L, L)] * cvo
                a3 = a3 + tbuf[slot, i + 1, pl.ds(3 * L, L)] * cvo
            return (a0, a1, a2, a3)

        for b in range(2):
            start_at(base + b * CHP, CHP, b)

        def outer_body(co_i, carry):
            for b in range(2):
                c = co_i * 2 + b
                wait_at(CHP, b)
                carry = lax.fori_loop(
                    0, CHP // L, functools.partial(group_body, slot=b), carry
                )

                @pl.when(c + 2 < NCH)
                def _():
                    start_at(base + (c + 2) * CHP, CHP, b)

            return carry

        z = jnp.zeros((L,), jnp.float32)
        carry = lax.fori_loop(0, NCH // 2, outer_body, (z, z, z, z))

        m = jnp.where(wid == 0, 1.0, 0.0).astype(jnp.float32)
        mv = jnp.full((L,), m, jnp.float32)
        for p0, n in TAIL:
            start_at(p0, n, 0)
            wait_at(n, 0)
            carry = lax.fori_loop(
                0,
                n // L,
                functools.partial(group_body, slot=0, scale=mv),
                carry,
            )

        a0, a1, a2, a3 = carry
        acc_v[pl.ds(0, L)] = a0
        acc_v[pl.ds(L, L)] = a1
        acc_v[pl.ds(2 * L, L)] = a2
        acc_v[pl.ds(3 * L, L)] = a3
        pltpu.sync_copy(acc_v, out_hbm.at[wid])

    return k(counts, table)


def _tc_finish(partials, w, b, n_rows, n_idx):
    def body(p_ref, w_ref, b_ref, o_ref):
        s = jnp.sum(p_ref[...], axis=0, keepdims=True) * (1.0 / n_idx)
        y = lax.dot_general(
            s, w_ref[...], (((1,), (1,)), ((), ())),
            preferred_element_type=jnp.float32,
        ) + b_ref[...][None, :]
        y = jnp.where(y >= 0, y, 0.01 * y)
        o_ref[...] = jnp.broadcast_to(y, o_ref.shape)

    return pl.pallas_call(
        body,
        out_shape=jax.ShapeDtypeStruct((n_rows, w.shape[0]), jnp.float32),
    )(partials, w, b)


@jax.jit
def kernel(x, mp_neighbors, item_table, neigh_w, neigh_b, mp):
    flat_idx = mp_neighbors.reshape(-1)
    counts = _sc_histogram(flat_idx, flat_idx.shape[0])
    partials = _sc_matvec(counts, item_table)
    return _tc_finish(
        partials, neigh_w, neigh_b, x.shape[0], flat_idx.shape[0]
    )


# speedup vs baseline: 1.6287x; 1.5209x over previous
import functools

import jax
import jax.numpy as jnp
from jax import lax
from jax.experimental import pallas as pl
from jax.experimental.pallas import tpu as pltpu
from jax.experimental.pallas import tpu_sc as plsc

NC = 2
NS = 16
NW = NC * NS
L = 16

VB = 1 << 20
SC_CHUNK = 128
TC_BLK = 16384


def _sc_histogram(idx, n_idx):
    per_tile = n_idx // NW
    nstream = per_tile // SC_CHUNK
    slice_per_tile = VB // NS
    mesh = plsc.VectorSubcoreMesh(core_axis_name="c", subcore_axis_name="s")

    @functools.partial(
        pl.kernel,
        out_type=jax.ShapeDtypeStruct((NC * VB,), jnp.float32),
        mesh=mesh,
        scratch_types=[
            pltpu.VMEM((per_tile,), jnp.int32),
            pltpu.VMEM((SC_CHUNK,), jnp.float32),
            pltpu.VMEM((slice_per_tile // 4,), jnp.float32),
            pltpu.VMEM_SHARED((VB,), jnp.float32),
            pltpu.SemaphoreType.DMA,
        ],
    )
    def k(idx_hbm, out_hbm, idx_v, ones_v, zero_v, hist_sp, sem):
        core = lax.axis_index("c")
        sub = lax.axis_index("s")
        base = (core * NS + sub) * per_tile
        pltpu.sync_copy(idx_hbm.at[pl.ds(base, per_tile)], idx_v)

        def fill_ones(kk, _):
            ones_v[pl.ds(kk * L, L)] = jnp.ones((L,), jnp.float32)
            return 0

        lax.fori_loop(0, SC_CHUNK // L, fill_ones, 0)

        def fill_zero(kk, _):
            zero_v[pl.ds(kk * L, L)] = jnp.zeros((L,), jnp.float32)
            return 0

        qtr = slice_per_tile // 4
        lax.fori_loop(0, qtr // L, fill_zero, 0, unroll=8)

        for q in range(4):
            pltpu.sync_copy(
                zero_v, hist_sp.at[pl.ds(sub * slice_per_tile + q * qtr, qtr)]
            )
        plsc.subcore_barrier()

        def fire(cc, _):
            pltpu.async_copy(
                ones_v,
                hist_sp.at[idx_v.at[pl.ds(cc * SC_CHUNK, SC_CHUNK)]],
                sem,
                add=True,
            )
            return 0

        lax.fori_loop(0, nstream, fire, 0)

        def drain(cc, _):
            pltpu.make_async_copy(
                ones_v,
                hist_sp.at[idx_v.at[pl.ds(0, SC_CHUNK)]],
                sem,
            ).wait()
            return 0

        lax.fori_loop(0, nstream, drain, 0)

        plsc.subcore_barrier()
        pltpu.sync_copy(
            hist_sp.at[pl.ds(sub * slice_per_tile, slice_per_tile)],
            out_hbm.at[pl.ds(core * VB + sub * slice_per_tile, slice_per_tile)],
        )

    return k(idx)


def _tc_matvec_finish(counts, table, w, b, n_rows, n_idx):
    v_rows = table.shape[0]
    d = table.shape[1]
    n_out = w.shape[0]
    nblk = (v_rows + TC_BLK - 1) // TC_BLK

    def body(ca_ref, cb_ref, t_ref, w_ref, b_ref, o_ref, acc_ref):
        i = pl.program_id(0)

        @pl.when(i == 0)
        def _():
            acc_ref[...] = jnp.zeros_like(acc_ref)

        c = (ca_ref[...] + cb_ref[...]).reshape(1, TC_BLK)
        acc_ref[...] += lax.dot_general(
            c, t_ref[...], (((1,), (0,)), ((), ())),
            preferred_element_type=jnp.float32,
        )

        @pl.when(i == nblk - 1)
        def _():
            s = acc_ref[...] * (1.0 / n_idx)
            y = lax.dot_general(
                s, w_ref[...], (((1,), (1,)), ((), ())),
                preferred_element_type=jnp.float32,
            ) + b_ref[...][None, :]
            y = jnp.where(y >= 0, y, 0.01 * y)
            o_ref[...] = jnp.broadcast_to(y, o_ref.shape)

    return pl.pallas_call(
        body,
        grid=(nblk,),
        in_specs=[
            pl.BlockSpec((TC_BLK,), lambda i: (i,)),
            pl.BlockSpec((TC_BLK,), lambda i: (VB // TC_BLK + i,)),
            pl.BlockSpec((TC_BLK, d), lambda i: (i, 0)),
            pl.BlockSpec((n_out, d), lambda i: (0, 0)),
            pl.BlockSpec((n_out,), lambda i: (0,)),
        ],
        out_specs=pl.BlockSpec((n_rows, n_out), lambda i: (0, 0)),
        out_shape=jax.ShapeDtypeStruct((n_rows, n_out), jnp.float32),
        scratch_shapes=[pltpu.VMEM((1, d), jnp.float32)],
    )(counts, counts, table, w, b)


@jax.jit
def kernel(x, mp_neighbors, item_table, neigh_w, neigh_b, mp):
    flat_idx = mp_neighbors.reshape(-1)
    counts = _sc_histogram(flat_idx, flat_idx.shape[0])
    return _tc_matvec_finish(
        counts, item_table, neigh_w, neigh_b, x.shape[0], flat_idx.shape[0]
    )


# speedup vs baseline: 1.6303x; 1.0010x over previous
import functools

import jax
import jax.numpy as jnp
from jax import lax
from jax.experimental import pallas as pl
from jax.experimental.pallas import tpu as pltpu
from jax.experimental.pallas import tpu_sc as plsc

NC = 2
NS = 16
NW = NC * NS
L = 16

VB = 1 << 20
SC_CHUNK = 128
TC_BLK = 32768


def _sc_histogram(idx, n_idx):
    per_tile = n_idx // NW
    nstream = per_tile // SC_CHUNK
    slice_per_tile = VB // NS
    mesh = plsc.VectorSubcoreMesh(core_axis_name="c", subcore_axis_name="s")

    @functools.partial(
        pl.kernel,
        out_type=jax.ShapeDtypeStruct((NC * VB,), jnp.float32),
        mesh=mesh,
        scratch_types=[
            pltpu.VMEM((per_tile,), jnp.int32),
            pltpu.VMEM((SC_CHUNK,), jnp.float32),
            pltpu.VMEM((slice_per_tile // 4,), jnp.float32),
            pltpu.VMEM_SHARED((VB,), jnp.float32),
            pltpu.SemaphoreType.DMA,
        ],
    )
    def k(idx_hbm, out_hbm, idx_v, ones_v, zero_v, hist_sp, sem):
        core = lax.axis_index("c")
        sub = lax.axis_index("s")
        base = (core * NS + sub) * per_tile
        pltpu.sync_copy(idx_hbm.at[pl.ds(base, per_tile)], idx_v)

        def fill_ones(kk, _):
            ones_v[pl.ds(kk * L, L)] = jnp.ones((L,), jnp.float32)
            return 0

        lax.fori_loop(0, SC_CHUNK // L, fill_ones, 0)

        def fill_zero(kk, _):
            zero_v[pl.ds(kk * L, L)] = jnp.zeros((L,), jnp.float32)
            return 0

        qtr = slice_per_tile // 4
        lax.fori_loop(0, qtr // L, fill_zero, 0, unroll=8)

        for q in range(4):
            pltpu.sync_copy(
                zero_v, hist_sp.at[pl.ds(sub * slice_per_tile + q * qtr, qtr)]
            )
        plsc.subcore_barrier()

        def fire(cc, _):
            pltpu.async_copy(
                ones_v,
                hist_sp.at[idx_v.at[pl.ds(cc * SC_CHUNK, SC_CHUNK)]],
                sem,
                add=True,
            )
            return 0

        lax.fori_loop(0, nstream, fire, 0)

        def drain(cc, _):
            pltpu.make_async_copy(
                ones_v,
                hist_sp.at[idx_v.at[pl.ds(0, SC_CHUNK)]],
                sem,
            ).wait()
            return 0

        lax.fori_loop(0, nstream, drain, 0)

        plsc.subcore_barrier()
        pltpu.sync_copy(
            hist_sp.at[pl.ds(sub * slice_per_tile, slice_per_tile)],
            out_hbm.at[pl.ds(core * VB + sub * slice_per_tile, slice_per_tile)],
        )

    return k(idx)


def _tc_matvec_finish(counts, table, w, b, n_rows, n_idx):
    v_rows = table.shape[0]
    d = table.shape[1]
    n_out = w.shape[0]
    nblk = (v_rows + TC_BLK - 1) // TC_BLK

    def body(ca_ref, cb_ref, t_ref, w_ref, b_ref, o_ref, acc_ref):
        i = pl.program_id(0)

        @pl.when(i == 0)
        def _():
            acc_ref[...] = jnp.zeros_like(acc_ref)

        c = (ca_ref[...] + cb_ref[...]).reshape(1, TC_BLK)
        acc_ref[...] += lax.dot_general(
            c, t_ref[...], (((1,), (0,)), ((), ())),
            preferred_element_type=jnp.float32,
        )

        @pl.when(i == nblk - 1)
        def _():
            s = acc_ref[...] * (1.0 / n_idx)
            y = lax.dot_general(
                s, w_ref[...], (((1,), (1,)), ((), ())),
                preferred_element_type=jnp.float32,
            ) + b_ref[...][None, :]
            y = jnp.where(y >= 0, y, 0.01 * y)
            o_ref[...] = jnp.broadcast_to(y, o_ref.shape)

    return pl.pallas_call(
        body,
        grid=(nblk,),
        in_specs=[
            pl.BlockSpec((TC_BLK,), lambda i: (i,)),
            pl.BlockSpec((TC_BLK,), lambda i: (VB // TC_BLK + i,)),
            pl.BlockSpec((TC_BLK, d), lambda i: (i, 0)),
            pl.BlockSpec((n_out, d), lambda i: (0, 0)),
            pl.BlockSpec((n_out,), lambda i: (0,)),
        ],
        out_specs=pl.BlockSpec((n_rows, n_out), lambda i: (0, 0)),
        out_shape=jax.ShapeDtypeStruct((n_rows, n_out), jnp.float32),
        scratch_shapes=[pltpu.VMEM((1, d), jnp.float32)],
    )(counts, counts, table, w, b)


@jax.jit
def kernel(x, mp_neighbors, item_table, neigh_w, neigh_b, mp):
    flat_idx = mp_neighbors.reshape(-1)
    counts = _sc_histogram(flat_idx, flat_idx.shape[0])
    return _tc_matvec_finish(
        counts, item_table, neigh_w, neigh_b, x.shape[0], flat_idx.shape[0]
    )
